# baseline (speedup 1.0000x reference)
import functools

import jax
import jax.numpy as jnp
from jax import lax
from jax.experimental import pallas as pl
from jax.experimental.pallas import tpu as pltpu
from jax.experimental.pallas import tpu_sc as plsc

_DTYPE_CODES = {
    jnp.dtype("int32"): 0,
    jnp.dtype("int64"): 1,
    jnp.dtype("uint32"): 2,
    jnp.dtype("uint64"): 3,
}


def _make_sc_kernel(obs_code: int, exp_code: int):
    mesh = plsc.VectorSubcoreMesh(core_axis_name="c", subcore_axis_name="s")

    @functools.partial(
        pl.kernel,
        mesh=mesh,
        out_type=jax.ShapeDtypeStruct((16,), jnp.int32),
        scratch_types=[pltpu.VMEM((16,), jnp.int32)],
    )
    def _sc_eq_kernel(out_hbm, vals_v):
        cid = lax.axis_index("c")
        sid = lax.axis_index("s")

        @pl.when(jnp.logical_and(cid == 0, sid == 0))
        def _():
            observed = jnp.full((16,), obs_code, dtype=jnp.int32)
            expected = jnp.full((16,), exp_code, dtype=jnp.int32)
            vals_v[...] = (observed == expected).astype(jnp.int32)
            pltpu.sync_copy(vals_v, out_hbm)

    return _sc_eq_kernel


def kernel(x):
    n = x.size

    def _branch_perm():
        return jax.random.permutation(jax.random.key(0), n)

    observed = jax.eval_shape(_branch_perm).dtype
    expected = jax.dtypes.canonicalize_dtype(jnp.dtype("int64"))

    obs_code = _DTYPE_CODES[jnp.dtype(observed)]
    exp_code = _DTYPE_CODES[jnp.dtype(expected)]
    out = _make_sc_kernel(obs_code, exp_code)()
    return out[:2].astype(bool)


# speedup vs baseline: 10.6868x; 10.6868x over previous
import jax
import jax.numpy as jnp
from jax.experimental import pallas as pl

_DTYPE_CODES = {
    jnp.dtype("int32"): 0,
    jnp.dtype("int64"): 1,
    jnp.dtype("uint32"): 2,
    jnp.dtype("uint64"): 3,
}


def _make_eq_kernel(obs_code: int, exp_code: int):
    def _eq_kernel(out_ref):
        observed = jnp.full((2,), obs_code, dtype=jnp.int32)
        expected = jnp.full((2,), exp_code, dtype=jnp.int32)
        out_ref[...] = observed == expected

    return _eq_kernel


def kernel(x):
    n = x.size

    def _branch_perm():
        return jax.random.permutation(jax.random.key(0), n)

    observed = jax.eval_shape(_branch_perm).dtype
    expected = jax.dtypes.canonicalize_dtype(jnp.dtype("int64"))

    obs_code = _DTYPE_CODES[jnp.dtype(observed)]
    exp_code = _DTYPE_CODES[jnp.dtype(expected)]
    return pl.pallas_call(
        _make_eq_kernel(obs_code, exp_code),
        out_shape=jax.ShapeDtypeStruct((2,), jnp.bool_),
    )()
